# Initial kernel scaffold; baseline (speedup 1.0000x reference)
import jax
import jax.numpy as jnp
from jax.experimental import pallas as pl


def kernel(text, emb_table, W, b):
    raise NotImplementedError("write your pallas kernel here")


# speedup vs baseline: 108.2068x; 108.2068x over previous
import functools

import jax
import jax.numpy as jnp
from jax import lax
from jax.experimental import pallas as pl
from jax.experimental.pallas import tpu as pltpu
from jax.experimental.pallas import tpu_sc as plsc

_NC = 2
_NS = 16
_NW = _NC * _NS
_LANES = 16

_CHUNK = 64


def _fold_table_kernel(emb_ref, w_ref, b_ref, p_ref):
    p_ref[:, :] = (
        jnp.sum(emb_ref[:, :] * w_ref[:, :], axis=1, keepdims=True)
        + b_ref[:, :]
    )


def _fold_table(emb_table, W, b):
    V, D = emb_table.shape
    RB = 5000
    return pl.pallas_call(
        _fold_table_kernel,
        grid=(V // RB,),
        in_specs=[
            pl.BlockSpec((RB, D), lambda i: (i, 0)),
            pl.BlockSpec((1, D), lambda i: (0, 0)),
            pl.BlockSpec((1, 1), lambda i: (0, 0)),
        ],
        out_specs=pl.BlockSpec((RB, 1), lambda i: (i, 0)),
        out_shape=jax.ShapeDtypeStruct((V, 1), jnp.float32),
    )(emb_table, W, b.reshape(1, 1))


def _make_sc_pool(V, B, S):
    nblk = B // _CHUNK
    blk_per_w = nblk // _NW
    groups = _CHUNK // _LANES
    inv_s = 1.0 / S
    mesh = plsc.VectorSubcoreMesh(
        core_axis_name="c", subcore_axis_name="s",
        num_cores=_NC, num_subcores=_NS,
    )

    @functools.partial(
        pl.kernel,
        out_type=jax.ShapeDtypeStruct((B,), jnp.float32),
        mesh=mesh,
        scratch_types=[
            pltpu.VMEM((V,), jnp.float32),
            pltpu.VMEM((S, _CHUNK), jnp.int32),
            pltpu.VMEM((_CHUNK,), jnp.float32),
        ],
        compiler_params=pltpu.CompilerParams(needs_layout_passes=False),
    )
    def sc_pool(p_hbm, idx_hbm, out_hbm, p_v, chunk_v, out_v):
        wid = lax.axis_index("s") * _NC + lax.axis_index("c")
        pltpu.sync_copy(p_hbm, p_v)

        def blk_body(i, carry):
            blk = wid * blk_per_w + i
            pltpu.sync_copy(idx_hbm.at[blk], chunk_v)

            def s_body(s, accs):
                out = []
                for g in range(groups):
                    ids = chunk_v[s, pl.ds(g * _LANES, _LANES)]
                    out.append(accs[g] + plsc.load_gather(p_v, [ids]))
                return tuple(out)

            zero = jnp.zeros((_LANES,), jnp.float32)
            accs = lax.fori_loop(0, S, s_body, (zero,) * groups)
            for g in range(groups):
                z = accs[g] * inv_s
                out_v[pl.ds(g * _LANES, _LANES)] = 1.0 / (1.0 + jnp.exp(-z))
            pltpu.sync_copy(out_v, out_hbm.at[pl.ds(blk * _CHUNK, _CHUNK)])
            return carry

        lax.fori_loop(0, blk_per_w, blk_body, 0)

    return sc_pool


def kernel(text, emb_table, W, b):
    B, S = text.shape
    V, _D = emb_table.shape
    p = _fold_table(emb_table, W, b).reshape(V)
    v_pad = -(-V // 128) * 128
    p = jnp.pad(p, (0, v_pad - V))
    idx_blocks = text.reshape(B // _CHUNK, _CHUNK, S).swapaxes(1, 2)
    out = _make_sc_pool(v_pad, B, S)(p, idx_blocks)
    return out.reshape(B, 1)
